# baseline (speedup 1.0000x reference)
import functools

import jax
import jax.numpy as jnp
from jax.experimental import pallas as pl
from jax.experimental.pallas import tpu as pltpu

N = 10000
NFEAT = 128
NHID = 16
NCLASS = 8

BM = 80
NB = N // BM
GROUP = 25
NG = NB // GROUP

_SLOPE = (1.0 / 8.0 + 1.0 / 3.0) / 2.0
_NS = NHID + NCLASS


def _round128(v):
    return min(-(-v // 128) * 128, N)


_WIDTHS = [_round128(BM * (GROUP * g + GROUP)) for g in range(NG)]


def _small_mm_kernel(x_ref, w1_ref, w2_ref, w4_ref, b2_ref, b4_ref,
                     s1_ref, skip0_ref, skip1_ref):
    x = x_ref[...]
    s1_ref[...] = jnp.dot(x, w1_ref[...], preferred_element_type=jnp.float32)
    skip0_ref[...] = (
        jnp.dot(x, w2_ref[...], preferred_element_type=jnp.float32)
        + b2_ref[...])
    skip1_ref[...] = (
        jnp.dot(x, w4_ref[...], preferred_element_type=jnp.float32)
        + b4_ref[...])


def _pass1_kernel(a_ref, s1_ref, skip0_ref, b1_ref, w3_ref,
                  skip1_ref, b3_ref, s2_ref, part_ref, s_s):
    i = pl.program_id(0)
    iblk = NB - 1 - i

    @pl.when(i == 0)
    def _():
        s_s[:, 0:NHID] = s1_ref[...]
        s_s[:, NHID:_NS] = jnp.zeros((N, NCLASS), jnp.float32)

    a = a_ref[...]
    r = jnp.dot(a, s_s[...], preferred_element_type=jnp.float32)
    h = r[:, 0:NHID] + b1_ref[...] + skip0_ref[...]
    h = jnp.where(h >= 0, h, _SLOPE * h)
    s2_blk = jnp.dot(h, w3_ref[...], preferred_element_type=jnp.float32)
    s_s[pl.ds(iblk * BM, BM), NHID:_NS] = s2_blk
    s2_ref[...] = s2_blk
    part_ref[...] = r[:, NHID:_NS] + b3_ref[...] + skip1_ref[...]


def _pass2_kernel(a_ref, s2_ref, part_ref, out_ref, *, g, w):
    i = pl.program_id(0)
    iblk = GROUP * g + i
    thresh = (iblk + 1) * BM
    row = jax.lax.broadcasted_iota(jnp.int32, (w, NCLASS), 0)
    s2m = jnp.where(row < thresh, s2_ref[...], 0.0)
    out_ref[...] = part_ref[...] + jnp.dot(
        a_ref[...], s2m, preferred_element_type=jnp.float32)


def kernel(x, adj, W1, b1, W2, b2, W3, b3, W4, b4):
    b1r = b1.reshape(1, NHID)
    b2r = b2.reshape(1, NHID)
    b3r = b3.reshape(1, NCLASS)
    b4r = b4.reshape(1, NCLASS)

    s1, skip0, skip1 = pl.pallas_call(
        _small_mm_kernel,
        out_shape=(
            jax.ShapeDtypeStruct((N, NHID), jnp.float32),
            jax.ShapeDtypeStruct((N, NHID), jnp.float32),
            jax.ShapeDtypeStruct((N, NCLASS), jnp.float32),
        ),
    )(x, W1, W2, W4, b2r, b4r)

    rev = lambda i: (NB - 1 - i, 0)
    s2, part = pl.pallas_call(
        _pass1_kernel,
        grid=(NB,),
        in_specs=[
            pl.BlockSpec((BM, N), rev),
            pl.BlockSpec((N, NHID), lambda i: (0, 0)),
            pl.BlockSpec((BM, NHID), rev),
            pl.BlockSpec((1, NHID), lambda i: (0, 0)),
            pl.BlockSpec((NHID, NCLASS), lambda i: (0, 0)),
            pl.BlockSpec((BM, NCLASS), rev),
            pl.BlockSpec((1, NCLASS), lambda i: (0, 0)),
        ],
        out_specs=(
            pl.BlockSpec((BM, NCLASS), rev),
            pl.BlockSpec((BM, NCLASS), rev),
        ),
        out_shape=(
            jax.ShapeDtypeStruct((N, NCLASS), jnp.float32),
            jax.ShapeDtypeStruct((N, NCLASS), jnp.float32),
        ),
        scratch_shapes=[pltpu.VMEM((N, _NS), jnp.float32)],
    )(adj, s1, skip0, b1r, W3, skip1, b3r)

    outs = []
    for g in range(NG):
        w = _WIDTHS[g]
        outs.append(pl.pallas_call(
            functools.partial(_pass2_kernel, g=g, w=w),
            grid=(GROUP,),
            in_specs=[
                pl.BlockSpec((BM, w), lambda i, g=g: (GROUP * g + i, 0)),
                pl.BlockSpec((w, NCLASS), lambda i: (0, 0)),
                pl.BlockSpec((BM, NCLASS), lambda i, g=g: (GROUP * g + i, 0)),
            ],
            out_specs=pl.BlockSpec((BM, NCLASS), lambda i: (i, 0)),
            out_shape=jax.ShapeDtypeStruct((GROUP * BM, NCLASS), jnp.float32),
        )(adj, s2, part))

    out = jnp.concatenate(outs, axis=0)
    return (out, W1, W2, W3, W4)


# speedup vs baseline: 1.3377x; 1.3377x over previous
import functools

import jax
import jax.numpy as jnp
from jax.experimental import pallas as pl
from jax.experimental.pallas import tpu as pltpu

N = 10000
NFEAT = 128
NHID = 16
NCLASS = 8

BM = 200
NB = N // BM
NP = NB // 2
GROUP = 10
NG = NB // GROUP

_SLOPE = (1.0 / 8.0 + 1.0 / 3.0) / 2.0
_NS = NHID + NCLASS


def _round128(v):
    return min(-(-v // 128) * 128, N)


_WIDTHS = [_round128(BM * (GROUP * g + GROUP)) for g in range(NG)]


def _small_mm_kernel(x_ref, w1_ref, w2_ref, w4_ref, b2_ref, b4_ref,
                     s1_ref, skip0_ref, skip1_ref):
    x = x_ref[...]
    s1_ref[...] = jnp.dot(x, w1_ref[...], preferred_element_type=jnp.float32)
    skip0_ref[...] = (
        jnp.dot(x, w2_ref[...], preferred_element_type=jnp.float32)
        + b2_ref[...])
    skip1_ref[...] = (
        jnp.dot(x, w4_ref[...], preferred_element_type=jnp.float32)
        + b4_ref[...])


def _pass1_kernel(alo_ref, ahi_ref, s1_ref, skip_ref, b1_ref, w3_ref,
                  skip1_ref, b3_ref, s2_ref, part_ref, s_s):
    i = pl.program_id(0)
    lo = NB - 2 - 2 * i

    @pl.when(i == 0)
    def _():
        s_s[:, 0:NHID] = s1_ref[...]
        s_s[:, NHID:_NS] = jnp.zeros((N, NCLASS), jnp.float32)

    r = jnp.dot(ahi_ref[...], s_s[...], preferred_element_type=jnp.float32)
    h = r[:, 0:NHID] + b1_ref[...] + skip_ref[BM:2 * BM, :]
    h = jnp.where(h >= 0, h, _SLOPE * h)
    s2_hi = jnp.dot(h, w3_ref[...], preferred_element_type=jnp.float32)
    s_s[pl.ds((lo + 1) * BM, BM), NHID:_NS] = s2_hi
    s2_ref[BM:2 * BM, :] = s2_hi
    part_ref[BM:2 * BM, :] = (
        r[:, NHID:_NS] + b3_ref[...] + skip1_ref[BM:2 * BM, :])

    r = jnp.dot(alo_ref[...], s_s[...], preferred_element_type=jnp.float32)
    h = r[:, 0:NHID] + b1_ref[...] + skip_ref[0:BM, :]
    h = jnp.where(h >= 0, h, _SLOPE * h)
    s2_lo = jnp.dot(h, w3_ref[...], preferred_element_type=jnp.float32)
    s_s[pl.ds(lo * BM, BM), NHID:_NS] = s2_lo
    s2_ref[0:BM, :] = s2_lo
    part_ref[0:BM, :] = (
        r[:, NHID:_NS] + b3_ref[...] + skip1_ref[0:BM, :])


def _pass2_kernel(a0_ref, a1_ref, s2_ref, part_ref, out_ref, *, g, w):
    i = pl.program_id(0)
    iblk0 = GROUP * g + 2 * i
    row = jax.lax.broadcasted_iota(jnp.int32, (w, NCLASS), 0)
    s2 = s2_ref[...]
    s2m0 = jnp.where(row < (iblk0 + 1) * BM, s2, 0.0)
    s2m1 = jnp.where(row < (iblk0 + 2) * BM, s2, 0.0)
    out_ref[0:BM, :] = part_ref[0:BM, :] + jnp.dot(
        a0_ref[...], s2m0, preferred_element_type=jnp.float32)
    out_ref[BM:2 * BM, :] = part_ref[BM:2 * BM, :] + jnp.dot(
        a1_ref[...], s2m1, preferred_element_type=jnp.float32)


def kernel(x, adj, W1, b1, W2, b2, W3, b3, W4, b4):
    b1r = b1.reshape(1, NHID)
    b2r = b2.reshape(1, NHID)
    b3r = b3.reshape(1, NCLASS)
    b4r = b4.reshape(1, NCLASS)

    s1, skip0, skip1 = pl.pallas_call(
        _small_mm_kernel,
        out_shape=(
            jax.ShapeDtypeStruct((N, NHID), jnp.float32),
            jax.ShapeDtypeStruct((N, NHID), jnp.float32),
            jax.ShapeDtypeStruct((N, NCLASS), jnp.float32),
        ),
    )(x, W1, W2, W4, b2r, b4r)

    revlo = lambda i: (NB - 2 - 2 * i, 0)
    revhi = lambda i: (NB - 1 - 2 * i, 0)
    revpair = lambda i: (NP - 1 - i, 0)
    s2, part = pl.pallas_call(
        _pass1_kernel,
        grid=(NP,),
        in_specs=[
            pl.BlockSpec((BM, N), revlo),
            pl.BlockSpec((BM, N), revhi),
            pl.BlockSpec((N, NHID), lambda i: (0, 0)),
            pl.BlockSpec((2 * BM, NHID), revpair),
            pl.BlockSpec((1, NHID), lambda i: (0, 0)),
            pl.BlockSpec((NHID, NCLASS), lambda i: (0, 0)),
            pl.BlockSpec((2 * BM, NCLASS), revpair),
            pl.BlockSpec((1, NCLASS), lambda i: (0, 0)),
        ],
        out_specs=(
            pl.BlockSpec((2 * BM, NCLASS), revpair),
            pl.BlockSpec((2 * BM, NCLASS), revpair),
        ),
        out_shape=(
            jax.ShapeDtypeStruct((N, NCLASS), jnp.float32),
            jax.ShapeDtypeStruct((N, NCLASS), jnp.float32),
        ),
        scratch_shapes=[pltpu.VMEM((N, _NS), jnp.float32)],
    )(adj, adj, s1, skip0, b1r, W3, skip1, b3r)

    outs = []
    for g in range(NG):
        w = _WIDTHS[g]
        outs.append(pl.pallas_call(
            functools.partial(_pass2_kernel, g=g, w=w),
            grid=(GROUP // 2,),
            in_specs=[
                pl.BlockSpec((BM, w), lambda i, g=g: (GROUP * g + 2 * i, 0)),
                pl.BlockSpec(
                    (BM, w), lambda i, g=g: (GROUP * g + 2 * i + 1, 0)),
                pl.BlockSpec((w, NCLASS), lambda i: (0, 0)),
                pl.BlockSpec(
                    (2 * BM, NCLASS), lambda i, g=g: (GROUP * g // 2 + i, 0)),
            ],
            out_specs=pl.BlockSpec((2 * BM, NCLASS), lambda i: (i, 0)),
            out_shape=jax.ShapeDtypeStruct((GROUP * BM, NCLASS), jnp.float32),
        )(adj, adj, s2, part))

    out = jnp.concatenate(outs, axis=0)
    return (out, W1, W2, W3, W4)


# speedup vs baseline: 1.5539x; 1.1617x over previous
import jax
import jax.numpy as jnp
from jax.experimental import pallas as pl
from jax.experimental.pallas import tpu as pltpu

N = 10000
NFEAT = 128
NHID = 16
NCLASS = 8

BM = 200
NB = N // BM
GROUP = 10
NG = NB // GROUP

_SLOPE = (1.0 / 8.0 + 1.0 / 3.0) / 2.0
_NS = NHID + NCLASS


def _round128(v):
    return min(-(-v // 128) * 128, N)


_WIDTHS = [_round128(BM * (GROUP * g + GROUP)) for g in range(NG)]


def _mega_kernel(x_ref, w1_ref, w2_ref, w3_ref, w4_ref,
                 b1_ref, b2_ref, b3_ref, b4_ref, adj_hbm,
                 out_ref, s_s, skip0_s, skip1_s, part_s):
    x = x_ref[...]
    s_s[:, 0:NHID] = jnp.dot(x, w1_ref[...],
                             preferred_element_type=jnp.float32)
    s_s[:, NHID:_NS] = jnp.zeros((N, NCLASS), jnp.float32)
    skip0_s[...] = (
        jnp.dot(x, w2_ref[...], preferred_element_type=jnp.float32)
        + b2_ref[...])
    skip1_s[...] = (
        jnp.dot(x, w4_ref[...], preferred_element_type=jnp.float32)
        + b4_ref[...])

    def p1_body(idx, a_ref):
        (i,) = idx
        iblk = NB - 1 - i
        r = jnp.dot(a_ref[...], s_s[...], preferred_element_type=jnp.float32)
        h = r[:, 0:NHID] + b1_ref[...] + skip0_s[pl.ds(iblk * BM, BM), :]
        h = jnp.where(h >= 0, h, _SLOPE * h)
        s2_blk = jnp.dot(h, w3_ref[...], preferred_element_type=jnp.float32)
        s_s[pl.ds(iblk * BM, BM), NHID:_NS] = s2_blk
        part_s[pl.ds(iblk * BM, BM), :] = (
            r[:, NHID:_NS] + b3_ref[...] + skip1_s[pl.ds(iblk * BM, BM), :])

    pltpu.emit_pipeline(
        p1_body,
        grid=(NB,),
        in_specs=[pl.BlockSpec((BM, N), lambda i: (NB - 1 - i, 0))],
        _explicit_indices=True,
    )(adj_hbm)

    for g in range(NG):
        w = _WIDTHS[g]

        def p2_body(idx, a_ref, g=g, w=w):
            (i,) = idx
            iblk = GROUP * g + i
            row = jax.lax.broadcasted_iota(jnp.int32, (w, NCLASS), 0)
            s2m = jnp.where(row < (iblk + 1) * BM, s_s[0:w, NHID:_NS], 0.0)
            out_ref[pl.ds(iblk * BM, BM), :] = (
                part_s[pl.ds(iblk * BM, BM), :]
                + jnp.dot(a_ref[...], s2m, preferred_element_type=jnp.float32))

        pltpu.emit_pipeline(
            p2_body,
            grid=(GROUP,),
            in_specs=[pl.BlockSpec((BM, w), lambda i, g=g: (GROUP * g + i, 0))],
            _explicit_indices=True,
        )(adj_hbm)


def kernel(x, adj, W1, b1, W2, b2, W3, b3, W4, b4):
    b1r = b1.reshape(1, NHID)
    b2r = b2.reshape(1, NHID)
    b3r = b3.reshape(1, NCLASS)
    b4r = b4.reshape(1, NCLASS)

    vm = pl.BlockSpec(memory_space=pltpu.MemorySpace.VMEM)
    out = pl.pallas_call(
        _mega_kernel,
        in_specs=[vm, vm, vm, vm, vm, vm, vm, vm, vm,
                  pl.BlockSpec(memory_space=pltpu.MemorySpace.HBM)],
        out_specs=pl.BlockSpec(memory_space=pltpu.MemorySpace.VMEM),
        out_shape=jax.ShapeDtypeStruct((N, NCLASS), jnp.float32),
        scratch_shapes=[
            pltpu.VMEM((N, _NS), jnp.float32),
            pltpu.VMEM((N, NHID), jnp.float32),
            pltpu.VMEM((N, NCLASS), jnp.float32),
            pltpu.VMEM((N, NCLASS), jnp.float32),
        ],
    )(x, W1, W2, W3, W4, b1r, b2r, b3r, b4r, adj)

    return (out, W1, W2, W3, W4)
